# baseline (speedup 1.0000x reference)
import functools

import jax
import jax.numpy as jnp
from jax import lax
from jax.experimental import pallas as pl
from jax.experimental.pallas import tpu as pltpu
from jax.experimental.pallas import tpu_sc as plsc

N = 10000
E = 320000
D = 128

NC = 2
NS = 16
NW = NC * NS
EW = E // NW
C = 80
NCHUNK = EW // C
NP = 10240
NZ = NP // NS
ZC = 80
NZP = NZ // ZC


def _bent_half(z):
    return (jnp.sqrt(z * z + 1.0) - 1.0) * 0.25 + z * 0.5


H1 = 192000
H2 = E - H1
GCC = 128


def _make_sc_gather(src_off, nedge):
    gnch = nedge // GCC
    gpt = gnch // NW
    grem = gnch - gpt * NW
    gpair = (gpt - 1) // 2
    gidx = gpt * GCC

    def body(node_hbm, src_hbm, out_hbm, idx_v, rows0, rows1, gsem, wsem):
        c = lax.axis_index("c")
        s = lax.axis_index("s")
        wid = s * NC + c
        base = wid * gpt * GCC
        rows = (rows0, rows1)

        pltpu.sync_copy(src_hbm.at[pl.ds(src_off + base, gidx)],
                        idx_v.at[pl.ds(0, gidx)])

        def fire(j, dst):
            return pltpu.async_copy(
                node_hbm.at[idx_v.at[pl.ds(j * GCC, GCC)]], dst, gsem)

        fire(0, rows0).wait()

        def pair(jj, carry):
            j0 = jj * 2
            for b in range(2):
                j = j0 + b
                w = pltpu.async_copy(
                    rows[b], out_hbm.at[pl.ds(base + j * GCC, GCC)], wsem)
                fire(j + 1, rows[1 - b]).wait()
                w.wait()
            return carry

        lax.fori_loop(0, gpair, pair, 0)
        if gpt % 2 == 0:
            w = pltpu.async_copy(
                rows0, out_hbm.at[pl.ds(base + (gpt - 2) * GCC, GCC)], wsem)
            fire(gpt - 1, rows1).wait()
            w.wait()
            pltpu.sync_copy(rows1, out_hbm.at[pl.ds(base + (gpt - 1) * GCC, GCC)])
        else:
            pltpu.sync_copy(rows0, out_hbm.at[pl.ds(base + (gpt - 1) * GCC, GCC)])

        @pl.when(wid < grem)
        def _rem():
            tail = (gpt * NW + wid) * GCC
            pltpu.sync_copy(src_hbm.at[pl.ds(src_off + tail, GCC)],
                            idx_v.at[pl.ds(gidx, GCC)])
            pltpu.async_copy(
                node_hbm.at[idx_v.at[pl.ds(gidx, GCC)]], rows0, gsem).wait()
            pltpu.sync_copy(rows0, out_hbm.at[pl.ds(tail, GCC)])

    return pl.kernel(
        body,
        out_type=jax.ShapeDtypeStruct((nedge, D), jnp.float32),
        mesh=plsc.VectorSubcoreMesh(core_axis_name="c", subcore_axis_name="s",
                                    num_cores=NC, num_subcores=NS),
        scratch_types=[
            pltpu.VMEM((gidx + GCC,), jnp.int32),
            pltpu.VMEM((GCC, D), jnp.float32),
            pltpu.VMEM((GCC, D), jnp.float32),
            pltpu.SemaphoreType.DMA,
            pltpu.SemaphoreType.DMA,
        ],
    )


_sc_gather_lo = _make_sc_gather(0, H1)
_sc_gather_hi = _make_sc_gather(H1, H2)


SCC = 128
NCHT = E // SCC
SPT = NCHT // NW
SREM = NCHT - SPT * NW
SPAIR = (SPT - 1) // 2


def _make_sc_scatter(dst_off):
    def body(y_hbm, dst_hbm, part_hbm, y0, y1, i0, i1, acc_sp,
             ysem, isem, ssem):
        c = lax.axis_index("c")
        s = lax.axis_index("s")
        wid = s * NC + c
        gbase = wid * SPT
        ybuf = (y0, y1)
        ibank = (i0, i1)

        def zrow(i, carry):
            for j in range(D // 16):
                y0[i, pl.ds(j * 16, 16)] = jnp.zeros((16,), jnp.float32)
            return carry

        lax.fori_loop(0, SCC, zrow, 0)
        for k in range(NZ // SCC):
            pltpu.sync_copy(y0, acc_sp.at[pl.ds(s * NZ + k * SCC, SCC)])
        plsc.subcore_barrier()

        pltpu.sync_copy(y_hbm.at[pl.ds(gbase * SCC, SCC)], y0)
        pltpu.sync_copy(dst_hbm.at[pl.ds(dst_off + gbase * SCC, SCC)], i0)

        def pair(jj, carry):
            j0 = jj * 2
            for b in range(2):
                j = j0 + b
                nxt = (gbase + j + 1) * SCC
                yd = pltpu.async_copy(y_hbm.at[pl.ds(nxt, SCC)],
                                      ybuf[1 - b], ysem)
                idd = pltpu.async_copy(dst_hbm.at[pl.ds(dst_off + nxt, SCC)],
                                       ibank[1 - b], isem)
                pltpu.async_copy(ybuf[b], acc_sp.at[ibank[b]], ssem,
                                 add=True).wait()
                yd.wait()
                idd.wait()
            return carry

        lax.fori_loop(0, SPAIR, pair, 0)
        if SPT % 2 == 0:
            nxt = (gbase + SPT - 1) * SCC
            yd = pltpu.async_copy(y_hbm.at[pl.ds(nxt, SCC)], y1, ysem)
            idd = pltpu.async_copy(dst_hbm.at[pl.ds(dst_off + nxt, SCC)],
                                   i1, isem)
            pltpu.async_copy(y0, acc_sp.at[i0], ssem, add=True).wait()
            yd.wait()
            idd.wait()
            pltpu.async_copy(y1, acc_sp.at[i1], ssem, add=True).wait()
        else:
            pltpu.async_copy(y0, acc_sp.at[i0], ssem, add=True).wait()
        @pl.when(wid < SREM)
        def _rem():
            tail = (SPT * NW + wid) * SCC
            pltpu.sync_copy(y_hbm.at[pl.ds(tail, SCC)], y1)
            pltpu.sync_copy(dst_hbm.at[pl.ds(dst_off + tail, SCC)], i1)
            pltpu.async_copy(y1, acc_sp.at[i1], ssem, add=True).wait()

        plsc.subcore_barrier()

        pltpu.sync_copy(acc_sp.at[pl.ds(s * NZ, NZ)],
                        part_hbm.at[c, pl.ds(s * NZ, NZ)])

    return pl.kernel(
        body,
        out_type=jax.ShapeDtypeStruct((NC, NP, D), jnp.float32),
        mesh=plsc.VectorSubcoreMesh(core_axis_name="c", subcore_axis_name="s",
                                    num_cores=NC, num_subcores=NS),
        scratch_types=[
            pltpu.VMEM((SCC, D), jnp.float32),
            pltpu.VMEM((SCC, D), jnp.float32),
            pltpu.VMEM((SCC,), jnp.int32),
            pltpu.VMEM((SCC,), jnp.int32),
            pltpu.VMEM_SHARED((NP, D), jnp.float32),
            pltpu.SemaphoreType.DMA,
            pltpu.SemaphoreType.DMA,
            pltpu.SemaphoreType.DMA,
        ],
    )


_sc_scatter = _make_sc_scatter(0)


BE = 8000
NBLK1 = H1 // BE
NBLK2 = H2 // BE


def _tc_edge_mlp_body(g_ref, e_ref, w_ref, b_ref, y_ref):
    z = jnp.dot(g_ref[...] + e_ref[...], w_ref[...],
                preferred_element_type=jnp.float32) + b_ref[...]
    y_ref[...] = _bent_half(z)


def _tc_edge_mlp_body_alias(g_ref, e_ref, w_ref, b_ref, _y_prev, y_ref):
    _tc_edge_mlp_body(g_ref, e_ref, w_ref, b_ref, y_ref)


def _tc_edge_mlp_half(g, edge_feats, w_e, b_e, blk_off, nblk, y_prev=None):
    args = [g, edge_feats, w_e, b_e]
    in_specs = [
        pl.BlockSpec((BE, D), lambda i: (i, 0)),
        pl.BlockSpec((BE, D), lambda i: (i + blk_off, 0)),
        pl.BlockSpec((D, D), lambda i: (0, 0)),
        pl.BlockSpec((1, D), lambda i: (0, 0)),
    ]
    kwargs = {}
    body = _tc_edge_mlp_body
    if y_prev is not None:
        args.append(y_prev)
        in_specs.append(pl.BlockSpec(memory_space=pl.ANY))
        kwargs["input_output_aliases"] = {4: 0}
        body = _tc_edge_mlp_body_alias
    return pl.pallas_call(
        body,
        grid=(nblk,),
        in_specs=in_specs,
        out_specs=pl.BlockSpec((BE, D), lambda i: (i + blk_off, 0)),
        out_shape=jax.ShapeDtypeStruct((E, D), jnp.float32),
        **kwargs,
    )(*args)


BN = 2000


def _tc_node_mlp_body(x_ref, pa0_ref, pa1_ref,
                      w1_ref, b1_ref, w2_ref, b2_ref, out_ref):
    agg = pa0_ref[0] + pa1_ref[0]
    x = x_ref[...] + agg * 0.1
    z1 = jnp.dot(x * 0.5, w1_ref[...], preferred_element_type=jnp.float32) \
        + b1_ref[...]
    h = (jnp.sqrt(z1 * z1 + 1.0) - 1.0) * 0.5 + z1
    z2 = jnp.dot(h, w2_ref[...], preferred_element_type=jnp.float32) \
        + b2_ref[...]
    out_ref[...] = (jnp.sqrt(z2 * z2 + 1.0) - 1.0) * 0.5 + z2


def _tc_node_mlp(node_feats, parts_a, w_a1, b_a1, w_a2, b_a2):
    return pl.pallas_call(
        _tc_node_mlp_body,
        grid=(N // BN,),
        in_specs=[
            pl.BlockSpec((BN, D), lambda i: (i, 0)),
            pl.BlockSpec((1, BN, D), lambda i: (0, i, 0)),
            pl.BlockSpec((1, BN, D), lambda i: (1, i, 0)),
            pl.BlockSpec((D, D), lambda i: (0, 0)),
            pl.BlockSpec((1, D), lambda i: (0, 0)),
            pl.BlockSpec((D, D), lambda i: (0, 0)),
            pl.BlockSpec((1, D), lambda i: (0, 0)),
        ],
        out_specs=pl.BlockSpec((BN, D), lambda i: (i, 0)),
        out_shape=jax.ShapeDtypeStruct((N, D), jnp.float32),
    )(node_feats, parts_a, parts_a, w_a1, b_a1, w_a2, b_a2)


def kernel(node_feats, edge_feats, edge_index, W_e, b_e, W_a1, b_a1, W_a2, b_a2):
    src = edge_index[0].astype(jnp.int32)
    dst = edge_index[1].astype(jnp.int32)
    g1 = _sc_gather_lo(node_feats, src)
    g2 = _sc_gather_hi(node_feats, src)
    y_lo = _tc_edge_mlp_half(g1, edge_feats, W_e, b_e.reshape(1, D), 0, NBLK1)
    y = _tc_edge_mlp_half(g2, edge_feats, W_e, b_e.reshape(1, D), NBLK1,
                          NBLK2, y_prev=y_lo)
    p1 = _sc_scatter(y, dst)
    x_out = _tc_node_mlp(node_feats, p1, W_a1, b_a1.reshape(1, D),
                         W_a2, b_a2.reshape(1, D))
    return (x_out, y)


# speedup vs baseline: 1.0089x; 1.0089x over previous
import functools

import jax
import jax.numpy as jnp
from jax import lax
from jax.experimental import pallas as pl
from jax.experimental.pallas import tpu as pltpu
from jax.experimental.pallas import tpu_sc as plsc

N = 10000
E = 320000
D = 128

NC = 2
NS = 16
NW = NC * NS
EW = E // NW
C = 80
NCHUNK = EW // C
NP = 10240
NZ = NP // NS
ZC = 80
NZP = NZ // ZC


def _bent_half(z):
    return (jnp.sqrt(z * z + 1.0) - 1.0) * 0.25 + z * 0.5


H1 = 160000
H2 = E - H1
GCC = 128


def _make_sc_gather(src_off, nedge):
    gnch = nedge // GCC
    gpt = gnch // NW
    grem = gnch - gpt * NW
    gpair = (gpt - 1) // 2
    gidx = gpt * GCC

    def body(node_hbm, src_hbm, out_hbm, idx_v, rows0, rows1, gsem, wsem):
        c = lax.axis_index("c")
        s = lax.axis_index("s")
        wid = s * NC + c
        base = wid * gpt * GCC
        rows = (rows0, rows1)

        pltpu.sync_copy(src_hbm.at[pl.ds(src_off + base, gidx)],
                        idx_v.at[pl.ds(0, gidx)])

        def fire(j, dst):
            return pltpu.async_copy(
                node_hbm.at[idx_v.at[pl.ds(j * GCC, GCC)]], dst, gsem)

        fire(0, rows0).wait()

        def pair(jj, carry):
            j0 = jj * 2
            for b in range(2):
                j = j0 + b
                w = pltpu.async_copy(
                    rows[b], out_hbm.at[pl.ds(base + j * GCC, GCC)], wsem)
                fire(j + 1, rows[1 - b]).wait()
                w.wait()
            return carry

        lax.fori_loop(0, gpair, pair, 0)
        if gpt % 2 == 0:
            w = pltpu.async_copy(
                rows0, out_hbm.at[pl.ds(base + (gpt - 2) * GCC, GCC)], wsem)
            fire(gpt - 1, rows1).wait()
            w.wait()
            pltpu.sync_copy(rows1, out_hbm.at[pl.ds(base + (gpt - 1) * GCC, GCC)])
        else:
            pltpu.sync_copy(rows0, out_hbm.at[pl.ds(base + (gpt - 1) * GCC, GCC)])

        @pl.when(wid < grem)
        def _rem():
            tail = (gpt * NW + wid) * GCC
            pltpu.sync_copy(src_hbm.at[pl.ds(src_off + tail, GCC)],
                            idx_v.at[pl.ds(gidx, GCC)])
            pltpu.async_copy(
                node_hbm.at[idx_v.at[pl.ds(gidx, GCC)]], rows0, gsem).wait()
            pltpu.sync_copy(rows0, out_hbm.at[pl.ds(tail, GCC)])

    return pl.kernel(
        body,
        out_type=jax.ShapeDtypeStruct((nedge, D), jnp.float32),
        mesh=plsc.VectorSubcoreMesh(core_axis_name="c", subcore_axis_name="s",
                                    num_cores=NC, num_subcores=NS),
        scratch_types=[
            pltpu.VMEM((gidx + GCC,), jnp.int32),
            pltpu.VMEM((GCC, D), jnp.float32),
            pltpu.VMEM((GCC, D), jnp.float32),
            pltpu.SemaphoreType.DMA,
            pltpu.SemaphoreType.DMA,
        ],
    )


_sc_gather_lo = _make_sc_gather(0, H1)
_sc_gather_hi = _make_sc_gather(H1, H2)


SCC = 128
NCHT = E // SCC
SPT = NCHT // NW
SREM = NCHT - SPT * NW
SPAIR = (SPT - 1) // 2


def _make_sc_scatter(dst_off):
    def body(y_hbm, dst_hbm, part_hbm, y0, y1, i0, i1, acc_sp,
             ysem, isem, ssem):
        c = lax.axis_index("c")
        s = lax.axis_index("s")
        wid = s * NC + c
        gbase = wid * SPT
        ybuf = (y0, y1)
        ibank = (i0, i1)

        def zrow(i, carry):
            for j in range(D // 16):
                y0[i, pl.ds(j * 16, 16)] = jnp.zeros((16,), jnp.float32)
            return carry

        lax.fori_loop(0, SCC, zrow, 0)
        for k in range(NZ // SCC):
            pltpu.sync_copy(y0, acc_sp.at[pl.ds(s * NZ + k * SCC, SCC)])
        plsc.subcore_barrier()

        pltpu.sync_copy(y_hbm.at[pl.ds(gbase * SCC, SCC)], y0)
        pltpu.sync_copy(dst_hbm.at[pl.ds(dst_off + gbase * SCC, SCC)], i0)

        def pair(jj, carry):
            j0 = jj * 2
            for b in range(2):
                j = j0 + b
                nxt = (gbase + j + 1) * SCC
                yd = pltpu.async_copy(y_hbm.at[pl.ds(nxt, SCC)],
                                      ybuf[1 - b], ysem)
                idd = pltpu.async_copy(dst_hbm.at[pl.ds(dst_off + nxt, SCC)],
                                       ibank[1 - b], isem)
                pltpu.async_copy(ybuf[b], acc_sp.at[ibank[b]], ssem,
                                 add=True).wait()
                yd.wait()
                idd.wait()
            return carry

        lax.fori_loop(0, SPAIR, pair, 0)
        if SPT % 2 == 0:
            nxt = (gbase + SPT - 1) * SCC
            yd = pltpu.async_copy(y_hbm.at[pl.ds(nxt, SCC)], y1, ysem)
            idd = pltpu.async_copy(dst_hbm.at[pl.ds(dst_off + nxt, SCC)],
                                   i1, isem)
            pltpu.async_copy(y0, acc_sp.at[i0], ssem, add=True).wait()
            yd.wait()
            idd.wait()
            pltpu.async_copy(y1, acc_sp.at[i1], ssem, add=True).wait()
        else:
            pltpu.async_copy(y0, acc_sp.at[i0], ssem, add=True).wait()
        @pl.when(wid < SREM)
        def _rem():
            tail = (SPT * NW + wid) * SCC
            pltpu.sync_copy(y_hbm.at[pl.ds(tail, SCC)], y1)
            pltpu.sync_copy(dst_hbm.at[pl.ds(dst_off + tail, SCC)], i1)
            pltpu.async_copy(y1, acc_sp.at[i1], ssem, add=True).wait()

        plsc.subcore_barrier()

        pltpu.sync_copy(acc_sp.at[pl.ds(s * NZ, NZ)],
                        part_hbm.at[c, pl.ds(s * NZ, NZ)])

    return pl.kernel(
        body,
        out_type=jax.ShapeDtypeStruct((NC, NP, D), jnp.float32),
        mesh=plsc.VectorSubcoreMesh(core_axis_name="c", subcore_axis_name="s",
                                    num_cores=NC, num_subcores=NS),
        scratch_types=[
            pltpu.VMEM((SCC, D), jnp.float32),
            pltpu.VMEM((SCC, D), jnp.float32),
            pltpu.VMEM((SCC,), jnp.int32),
            pltpu.VMEM((SCC,), jnp.int32),
            pltpu.VMEM_SHARED((NP, D), jnp.float32),
            pltpu.SemaphoreType.DMA,
            pltpu.SemaphoreType.DMA,
            pltpu.SemaphoreType.DMA,
        ],
    )


_sc_scatter = _make_sc_scatter(0)


BE = 8000
NBLK1 = H1 // BE
NBLK2 = H2 // BE


def _tc_edge_mlp_body(g_ref, e_ref, w_ref, b_ref, y_ref):
    z = jnp.dot(g_ref[...] + e_ref[...], w_ref[...],
                preferred_element_type=jnp.float32) + b_ref[...]
    y_ref[...] = _bent_half(z)


def _tc_edge_mlp_body_alias(g_ref, e_ref, w_ref, b_ref, _y_prev, y_ref):
    _tc_edge_mlp_body(g_ref, e_ref, w_ref, b_ref, y_ref)


def _tc_edge_mlp_half(g, edge_feats, w_e, b_e, blk_off, nblk, y_prev=None):
    args = [g, edge_feats, w_e, b_e]
    in_specs = [
        pl.BlockSpec((BE, D), lambda i: (i, 0)),
        pl.BlockSpec((BE, D), lambda i: (i + blk_off, 0)),
        pl.BlockSpec((D, D), lambda i: (0, 0)),
        pl.BlockSpec((1, D), lambda i: (0, 0)),
    ]
    kwargs = {}
    body = _tc_edge_mlp_body
    if y_prev is not None:
        args.append(y_prev)
        in_specs.append(pl.BlockSpec(memory_space=pl.ANY))
        kwargs["input_output_aliases"] = {4: 0}
        body = _tc_edge_mlp_body_alias
    return pl.pallas_call(
        body,
        grid=(nblk,),
        in_specs=in_specs,
        out_specs=pl.BlockSpec((BE, D), lambda i: (i + blk_off, 0)),
        out_shape=jax.ShapeDtypeStruct((E, D), jnp.float32),
        **kwargs,
    )(*args)


BN = 2000


def _tc_node_mlp_body(x_ref, pa0_ref, pa1_ref,
                      w1_ref, b1_ref, w2_ref, b2_ref, out_ref):
    agg = pa0_ref[0] + pa1_ref[0]
    x = x_ref[...] + agg * 0.1
    z1 = jnp.dot(x * 0.5, w1_ref[...], preferred_element_type=jnp.float32) \
        + b1_ref[...]
    h = (jnp.sqrt(z1 * z1 + 1.0) - 1.0) * 0.5 + z1
    z2 = jnp.dot(h, w2_ref[...], preferred_element_type=jnp.float32) \
        + b2_ref[...]
    out_ref[...] = (jnp.sqrt(z2 * z2 + 1.0) - 1.0) * 0.5 + z2


def _tc_node_mlp(node_feats, parts_a, w_a1, b_a1, w_a2, b_a2):
    return pl.pallas_call(
        _tc_node_mlp_body,
        grid=(N // BN,),
        in_specs=[
            pl.BlockSpec((BN, D), lambda i: (i, 0)),
            pl.BlockSpec((1, BN, D), lambda i: (0, i, 0)),
            pl.BlockSpec((1, BN, D), lambda i: (1, i, 0)),
            pl.BlockSpec((D, D), lambda i: (0, 0)),
            pl.BlockSpec((1, D), lambda i: (0, 0)),
            pl.BlockSpec((D, D), lambda i: (0, 0)),
            pl.BlockSpec((1, D), lambda i: (0, 0)),
        ],
        out_specs=pl.BlockSpec((BN, D), lambda i: (i, 0)),
        out_shape=jax.ShapeDtypeStruct((N, D), jnp.float32),
    )(node_feats, parts_a, parts_a, w_a1, b_a1, w_a2, b_a2)


def kernel(node_feats, edge_feats, edge_index, W_e, b_e, W_a1, b_a1, W_a2, b_a2):
    src = edge_index[0].astype(jnp.int32)
    dst = edge_index[1].astype(jnp.int32)
    g1 = _sc_gather_lo(node_feats, src)
    g2 = _sc_gather_hi(node_feats, src)
    y_lo = _tc_edge_mlp_half(g1, edge_feats, W_e, b_e.reshape(1, D), 0, NBLK1)
    y = _tc_edge_mlp_half(g2, edge_feats, W_e, b_e.reshape(1, D), NBLK1,
                          NBLK2, y_prev=y_lo)
    p1 = _sc_scatter(y, dst)
    x_out = _tc_node_mlp(node_feats, p1, W_a1, b_a1.reshape(1, D),
                         W_a2, b_a2.reshape(1, D))
    return (x_out, y)
